# baseline (speedup 1.0000x reference)
import jax
import jax.numpy as jnp
import numpy as np
from jax.experimental import pallas as pl
from jax.experimental.pallas import tpu as pltpu

N = 16384
C = 1000
K = int(N * 0.7)
BR = 512
NB = N // BR


def _ce_kernel(pred_ref, tgt_ref, ce_ref):
    x = pred_ref[...]
    tgt = tgt_ref[0]
    m = jnp.max(x, axis=1, keepdims=True)
    s = jnp.sum(jnp.exp(x - m), axis=1, keepdims=True)
    col = jax.lax.broadcasted_iota(jnp.int32, (BR, C), 1)
    tv = jnp.sum(jnp.where(col == tgt, x, 0.0), axis=1, keepdims=True)
    ce_ref[...] = jnp.log(s) + m - tv


def _select_kernel(ce_ref, out_ref):
    x = ce_ref[...]
    bits = jax.lax.bitcast_convert_type(x, jnp.int32)
    hi0 = jnp.max(bits)

    def body(_, carry):
        lo, hi = carry
        mid = lo + ((hi - lo + 1) >> 1)
        cnt = jnp.sum((bits >= mid).astype(jnp.int32))
        ok = cnt >= K
        return jnp.where(ok, mid, lo), jnp.where(ok, hi, mid - 1)

    lo, _ = jax.lax.fori_loop(0, 31, body, (jnp.int32(0), hi0))
    tval = jax.lax.bitcast_convert_type(lo, jnp.float32)
    gt = bits > lo
    cnt_gt = jnp.sum(gt.astype(jnp.int32))
    sum_gt = jnp.sum(jnp.where(gt, x, 0.0))
    res = (sum_gt + (K - cnt_gt).astype(jnp.float32) * tval) / np.float32(K)
    out_ref[...] = jnp.reshape(res, (1, 1))


def kernel(pred, target):
    tgt = target.astype(jnp.int32).reshape(NB, BR, 1)
    ce = pl.pallas_call(
        _ce_kernel,
        grid=(NB,),
        in_specs=[
            pl.BlockSpec((BR, C), lambda i: (i, 0)),
            pl.BlockSpec((1, BR, 1), lambda i: (i, 0, 0)),
        ],
        out_specs=pl.BlockSpec((BR, 1), lambda i: (i, 0)),
        out_shape=jax.ShapeDtypeStruct((N, 1), jnp.float32),
    )(pred, tgt)
    out = pl.pallas_call(
        _select_kernel,
        out_shape=jax.ShapeDtypeStruct((1, 1), jnp.float32),
    )(ce.reshape(128, 128))
    return out[0, 0]


# speedup vs baseline: 1.1568x; 1.1568x over previous
import jax
import jax.numpy as jnp
import numpy as np
from jax.experimental import pallas as pl
from jax.experimental.pallas import tpu as pltpu

N = 16384
C = 1000
K = int(N * 0.7)
BR = 2048
NB = N // BR


def _ce_kernel(pred_ref, tgt_ref, ce_ref):
    x = pred_ref[...]
    tgt = tgt_ref[0]
    m = jnp.max(x, axis=1, keepdims=True)
    s = jnp.sum(jnp.exp(x - m), axis=1, keepdims=True)
    col = jax.lax.broadcasted_iota(jnp.int32, (BR, C), 1)
    tv = jnp.sum(jnp.where(col == tgt, x, 0.0), axis=1, keepdims=True)
    ce_ref[...] = jnp.log(s) + m - tv


def _select_kernel(ce_ref, out_ref):
    x = ce_ref[...]
    bits = jax.lax.bitcast_convert_type(x, jnp.int32)
    hi0 = jnp.max(bits)

    def body(_, carry):
        lo, hi = carry
        mid = lo + ((hi - lo + 1) >> 1)
        cnt = jnp.sum((bits >= mid).astype(jnp.int32))
        ok = cnt >= K
        return jnp.where(ok, mid, lo), jnp.where(ok, hi, mid - 1)

    lo, _ = jax.lax.fori_loop(0, 31, body, (jnp.int32(0), hi0))
    tval = jax.lax.bitcast_convert_type(lo, jnp.float32)
    gt = bits > lo
    cnt_gt = jnp.sum(gt.astype(jnp.int32))
    sum_gt = jnp.sum(jnp.where(gt, x, 0.0))
    res = (sum_gt + (K - cnt_gt).astype(jnp.float32) * tval) / np.float32(K)
    out_ref[...] = jnp.reshape(res, (1, 1))


def kernel(pred, target):
    tgt = target.astype(jnp.int32).reshape(NB, BR, 1)
    ce = pl.pallas_call(
        _ce_kernel,
        grid=(NB,),
        in_specs=[
            pl.BlockSpec((BR, C), lambda i: (i, 0)),
            pl.BlockSpec((1, BR, 1), lambda i: (i, 0, 0)),
        ],
        out_specs=pl.BlockSpec((BR, 1), lambda i: (i, 0)),
        out_shape=jax.ShapeDtypeStruct((N, 1), jnp.float32),
    )(pred, tgt)
    out = pl.pallas_call(
        _select_kernel,
        out_shape=jax.ShapeDtypeStruct((1, 1), jnp.float32),
    )(ce.reshape(128, 128))
    return out[0, 0]
